# baseline (speedup 1.0000x reference)
import functools

import jax
import jax.numpy as jnp
from jax import lax
from jax.experimental import pallas as pl
from jax.experimental.pallas import tpu as pltpu
from jax.experimental.pallas import tpu_sc as plsc

_DIM = 1024
_SEQ = 8192
_SCALE = _DIM ** (-0.5)
_NC = 2
_NS = 16
_NW = _NC * _NS
_RPW = _SEQ // _NW
_CHUNK = 32
_NCHUNKS = _RPW // _CHUNK
_NBUF = 3
_LANES = 16

_mesh = plsc.VectorSubcoreMesh(core_axis_name="c", subcore_axis_name="s")


@functools.partial(
    pl.kernel,
    out_type=jax.ShapeDtypeStruct((_SEQ, _DIM), jnp.float32),
    mesh=_mesh,
    scratch_types=[
        pltpu.VMEM((_NBUF, _CHUNK, _DIM), jnp.float32),
        [pltpu.SemaphoreType.DMA] * _NBUF,
        [pltpu.SemaphoreType.DMA] * _NBUF,
    ],
)
def _sc_scale(emb_hbm, out_hbm, buf, sin, sout):
    wid = lax.axis_index("s") * _NC + lax.axis_index("c")
    base = wid * _RPW

    def start_in(c):
        b = c % _NBUF
        return pltpu.async_copy(
            emb_hbm.at[pl.ds(base + c * _CHUNK, _CHUNK)], buf.at[b], sin[b])

    def start_out(c):
        b = c % _NBUF
        return pltpu.async_copy(
            buf.at[b], out_hbm.at[pl.ds(base + c * _CHUNK, _CHUNK)], sout[b])

    def scale_buf(b):
        def body(r, carry):
            for j in range(_DIM // _LANES):
                idx = (b, r, pl.ds(j * _LANES, _LANES))
                buf[idx] = buf[idx] * _SCALE
            return carry
        lax.fori_loop(0, _CHUNK, body, 0, unroll=False)

    d_in = {c: start_in(c) for c in range(min(2, _NCHUNKS))}
    d_out = {}
    for c in range(_NCHUNKS):
        d_in[c].wait()
        if c + 2 < _NCHUNKS:
            if c >= 1:
                d_out[c - 1].wait()
            d_in[c + 2] = start_in(c + 2)
        scale_buf(c % _NBUF)
        d_out[c] = start_out(c)
    for c in range(max(0, _NCHUNKS - 3), _NCHUNKS):
        d_out[c].wait()


def kernel(x, emb):
    seq_len = x.shape[1]
    return _sc_scale(emb[:seq_len])
